# Initial kernel scaffold; baseline (speedup 1.0000x reference)
import jax
import jax.numpy as jnp
from jax.experimental import pallas as pl


def kernel(positions, pe_k_weight):
    raise NotImplementedError("write your pallas kernel here")


# speedup vs baseline: 4.7596x; 4.7596x over previous
import functools

import jax
import jax.numpy as jnp
from jax import lax
from jax.experimental import pallas as pl
from jax.experimental.pallas import tpu as pltpu
from jax.experimental.pallas import tpu_sc as plsc

_MAXLEN = 2048
_EMB = 64
_NC, _NS = 2, 16
_NW = _NC * _NS
_CHUNK = 512
_SUB = 128
_NSUB = _CHUNK // _SUB


@functools.cache
def _make_sc_gather(B: int):
    bpw = B // _NW
    nchunk = bpw // _CHUNK
    mesh = plsc.VectorSubcoreMesh(
        core_axis_name="c", subcore_axis_name="s",
        num_cores=_NC, num_subcores=_NS,
    )

    @functools.partial(
        pl.kernel,
        out_type=jax.ShapeDtypeStruct((B, _EMB), jnp.float32),
        mesh=mesh,
        compiler_params=pltpu.CompilerParams(use_tc_tiling_on_sc=False),
        scratch_types=[
            pltpu.VMEM((_CHUNK,), jnp.int32),
            pltpu.VMEM((_NSUB, _SUB), jnp.int32),
            pltpu.VMEM((_CHUNK, _EMB), jnp.float32),
            pltpu.SemaphoreType.DMA,
        ],
    )
    def gather_kernel(pos_hbm, tab_hbm, out_hbm, idx_raw, idx2, rows, sem):
        wid = lax.axis_index("s") * _NC + lax.axis_index("c")
        base = wid * bpw

        def body(g, carry):
            off = base + g * _CHUNK
            pltpu.sync_copy(pos_hbm.at[pl.ds(off, _CHUNK)], idx_raw)
            for i in range(_CHUNK // 16):
                v = idx_raw[pl.ds(i * 16, 16)]
                v = jnp.minimum(v, _MAXLEN - 1)
                v = jnp.maximum(v, -_MAXLEN)
                v = v + _MAXLEN
                r, c = divmod(i * 16, _SUB)
                idx2[r, pl.ds(c, 16)] = v
            copies = [
                pltpu.async_copy(
                    tab_hbm.at[idx2.at[j]],
                    rows.at[pl.ds(j * _SUB, _SUB)],
                    sem,
                )
                for j in range(_NSUB)
            ]
            for cp in copies:
                cp.wait()
            pltpu.sync_copy(rows, out_hbm.at[pl.ds(off, _CHUNK)])
            return carry

        lax.fori_loop(0, nchunk, body, 0)

    return gather_kernel


@jax.jit
def kernel(positions, pe_k_weight):
    seq_a, seq_b = positions.shape
    B = seq_a * seq_b
    pos_flat = positions.reshape(B).astype(jnp.int32)
    out = _make_sc_gather(B)(pos_flat, pe_k_weight)
    return out.reshape(seq_a, seq_b, _EMB)


# speedup vs baseline: 4.8375x; 1.0164x over previous
import functools

import jax
import jax.numpy as jnp
from jax import lax
from jax.experimental import pallas as pl
from jax.experimental.pallas import tpu as pltpu
from jax.experimental.pallas import tpu_sc as plsc

_MAXLEN = 2048
_EMB = 64
_NC, _NS = 2, 16
_NW = _NC * _NS
_CHUNK = 512
_SUB = 128
_NSUB = _CHUNK // _SUB
_NBUF = 2


@functools.cache
def _make_sc_gather(B: int):
    bpw = B // _NW
    nchunk = bpw // _CHUNK
    assert nchunk % _NBUF == 0
    mesh = plsc.VectorSubcoreMesh(
        core_axis_name="c", subcore_axis_name="s",
        num_cores=_NC, num_subcores=_NS,
    )

    @functools.partial(
        pl.kernel,
        out_type=jax.ShapeDtypeStruct((B, _EMB), jnp.float32),
        mesh=mesh,
        compiler_params=pltpu.CompilerParams(use_tc_tiling_on_sc=False),
        scratch_types=[
            pltpu.VMEM((_NBUF, _CHUNK), jnp.int32),
            pltpu.VMEM((_NBUF, _NSUB, _SUB), jnp.int32),
            pltpu.VMEM((_NBUF, _CHUNK, _EMB), jnp.float32),
            pltpu.SemaphoreType.DMA((_NBUF,)),
            pltpu.SemaphoreType.DMA((_NBUF,)),
            pltpu.SemaphoreType.DMA((_NBUF,)),
        ],
    )
    def gather_kernel(pos_hbm, tab_hbm, out_hbm, idx_raw, idx2, rows,
                      isem, gsem, osem):
        wid = lax.axis_index("s") * _NC + lax.axis_index("c")
        base = wid * bpw

        def idx_copy(g, b):
            return pltpu.make_async_copy(
                pos_hbm.at[pl.ds(base + g * _CHUNK, _CHUNK)],
                idx_raw.at[b], isem.at[b])

        def gat_copy(b, j):
            return pltpu.make_async_copy(
                tab_hbm.at[idx2.at[b, j]],
                rows.at[b, pl.ds(j * _SUB, _SUB)], gsem.at[b])

        def out_copy(g, b):
            return pltpu.make_async_copy(
                rows.at[b],
                out_hbm.at[pl.ds(base + g * _CHUNK, _CHUNK)], osem.at[b])

        for b in range(_NBUF):
            idx_copy(b, b).start()

        def pair_body(p, carry):
            for b in range(_NBUF):
                g = p * _NBUF + b
                idx_copy(g, b).wait()
                for i in range(_CHUNK // 16):
                    v = idx_raw[b, pl.ds(i * 16, 16)]
                    v = jnp.minimum(v, _MAXLEN - 1)
                    v = jnp.maximum(v, -_MAXLEN)
                    v = v + _MAXLEN
                    r, c = divmod(i * 16, _SUB)
                    idx2[b, r, pl.ds(c, 16)] = v

                @pl.when(p > 0)
                def _():
                    out_copy(g - _NBUF, b).wait()

                for j in range(_NSUB):
                    gat_copy(b, j).start()

                @pl.when(g + _NBUF < nchunk)
                def _():
                    idx_copy(g + _NBUF, b).start()

                for j in range(_NSUB):
                    gat_copy(b, j).wait()
                out_copy(g, b).start()
            return carry

        lax.fori_loop(0, nchunk // _NBUF, pair_body, 0)
        for b in range(_NBUF):
            out_copy(nchunk - _NBUF + b, b).wait()

    return gather_kernel


@jax.jit
def kernel(positions, pe_k_weight):
    seq_a, seq_b = positions.shape
    B = seq_a * seq_b
    pos_flat = positions.reshape(B).astype(jnp.int32)
    out = _make_sc_gather(B)(pos_flat, pe_k_weight)
    return out.reshape(seq_a, seq_b, _EMB)


# speedup vs baseline: 6.1753x; 1.2765x over previous
import functools

import jax
import jax.numpy as jnp
from jax import lax
from jax.experimental import pallas as pl
from jax.experimental.pallas import tpu as pltpu
from jax.experimental.pallas import tpu_sc as plsc

_MAXLEN = 2048
_EMB = 64
_NC, _NS = 2, 16
_NW = _NC * _NS
_CHUNK = 512
_SUB = 128
_NSUB = _CHUNK // _SUB
_NBUF = 2


@functools.cache
def _make_sc_gather(B: int):
    bpw = B // _NW
    nchunk = bpw // _CHUNK
    assert nchunk % _NBUF == 0
    mesh = plsc.VectorSubcoreMesh(
        core_axis_name="c", subcore_axis_name="s",
        num_cores=_NC, num_subcores=_NS,
    )

    @functools.partial(
        pl.kernel,
        out_type=jax.ShapeDtypeStruct((B, _EMB), jnp.float32),
        mesh=mesh,
        compiler_params=pltpu.CompilerParams(use_tc_tiling_on_sc=False),
        scratch_types=[
            pltpu.VMEM((_NBUF, _CHUNK), jnp.int32),
            pltpu.VMEM((_NBUF, _NSUB, _SUB), jnp.int32),
            pltpu.VMEM((_NBUF, _CHUNK, _EMB), jnp.float32),
            pltpu.VMEM_SHARED((2 * _MAXLEN, _EMB), jnp.float32),
            pltpu.SemaphoreType.DMA((_NBUF,)),
            pltpu.SemaphoreType.DMA((_NBUF,)),
            pltpu.SemaphoreType.DMA((_NBUF,)),
        ],
    )
    def gather_kernel(pos_hbm, tab_hbm, out_hbm, idx_raw, idx2, rows,
                      tab_sh, isem, gsem, osem):
        wid = lax.axis_index("s") * _NC + lax.axis_index("c")
        base = wid * bpw

        @pl.when(lax.axis_index("s") == 0)
        def _():
            pltpu.sync_copy(tab_hbm, tab_sh)
        plsc.subcore_barrier()

        def idx_copy(g, b):
            return pltpu.make_async_copy(
                pos_hbm.at[pl.ds(base + g * _CHUNK, _CHUNK)],
                idx_raw.at[b], isem.at[b])

        def gat_copy(b, j):
            return pltpu.make_async_copy(
                tab_sh.at[idx2.at[b, j]],
                rows.at[b, pl.ds(j * _SUB, _SUB)], gsem.at[b])

        def out_copy(g, b):
            return pltpu.make_async_copy(
                rows.at[b],
                out_hbm.at[pl.ds(base + g * _CHUNK, _CHUNK)], osem.at[b])

        for b in range(_NBUF):
            idx_copy(b, b).start()

        def pair_body(p, carry):
            for b in range(_NBUF):
                g = p * _NBUF + b
                idx_copy(g, b).wait()
                for i in range(_CHUNK // 16):
                    v = idx_raw[b, pl.ds(i * 16, 16)]
                    v = jnp.minimum(v, _MAXLEN - 1)
                    v = jnp.maximum(v, -_MAXLEN)
                    v = v + _MAXLEN
                    r, c = divmod(i * 16, _SUB)
                    idx2[b, r, pl.ds(c, 16)] = v

                @pl.when(p > 0)
                def _():
                    out_copy(g - _NBUF, b).wait()

                for j in range(_NSUB):
                    gat_copy(b, j).start()

                @pl.when(g + _NBUF < nchunk)
                def _():
                    idx_copy(g + _NBUF, b).start()

                for j in range(_NSUB):
                    gat_copy(b, j).wait()
                out_copy(g, b).start()
            return carry

        lax.fori_loop(0, nchunk // _NBUF, pair_body, 0)
        for b in range(_NBUF):
            out_copy(nchunk - _NBUF + b, b).wait()

    return gather_kernel


@jax.jit
def kernel(positions, pe_k_weight):
    seq_a, seq_b = positions.shape
    B = seq_a * seq_b
    pos_flat = positions.reshape(B).astype(jnp.int32)
    out = _make_sc_gather(B)(pos_flat, pe_k_weight)
    return out.reshape(seq_a, seq_b, _EMB)
